# Initial kernel scaffold; baseline (speedup 1.0000x reference)
import jax
import jax.numpy as jnp
from jax.experimental import pallas as pl


def kernel(x, P, CB):
    raise NotImplementedError("write your pallas kernel here")


# speedup vs baseline: 2.3209x; 2.3209x over previous
import jax
import jax.numpy as jnp
from jax import lax
from jax.experimental import pallas as pl

_B, _T, _D = 8, 1024, 1024
_CD, _V = 16, 8192
_TB = 256


def _vq_body(x_ref, p_ref, cbt_ref, out_ref):
    xp = lax.dot_general(
        x_ref[...], p_ref[...], (((1,), (0,)), ((), ())),
        preferred_element_type=jnp.float32,
    )
    norm = jnp.sqrt(jnp.sum(xp * xp, axis=1, keepdims=True))
    xn = xp / jnp.clip(norm, 1e-12, None)
    cbt = cbt_ref[...]
    csq = jnp.sum(cbt * cbt, axis=0, keepdims=True)
    dots = lax.dot_general(
        xn, cbt, (((1,), (0,)), ((), ())),
        preferred_element_type=jnp.float32,
    )
    s = csq - 2.0 * dots
    out_ref[...] = jnp.argmin(s, axis=1).astype(jnp.int32)[:, None]


def kernel(x, P, CB):
    x2 = x.reshape(_B * _T, _D)
    cbt = CB.T
    out = pl.pallas_call(
        _vq_body,
        grid=(_B * _T // _TB,),
        in_specs=[
            pl.BlockSpec((_TB, _D), lambda i: (i, 0)),
            pl.BlockSpec((_D, _CD), lambda i: (0, 0)),
            pl.BlockSpec((_CD, _V), lambda i: (0, 0)),
        ],
        out_specs=pl.BlockSpec((_TB, 1), lambda i: (i, 0)),
        out_shape=jax.ShapeDtypeStruct((_B * _T, 1), jnp.int32),
    )(x2, P, cbt)
    return out.reshape(_B, _T)


# speedup vs baseline: 2.4875x; 1.0718x over previous
import jax
import jax.numpy as jnp
from jax import lax
from jax.experimental import pallas as pl

_B, _T, _D = 8, 1024, 1024
_CD, _V = 16, 8192
_TB = 256


def _vq_body(x_ref, p_ref, cbt_ref, out_ref):
    xp = lax.dot_general(
        x_ref[...], p_ref[...], (((1,), (0,)), ((), ())),
        preferred_element_type=jnp.float32,
    )
    norm = jnp.sqrt(jnp.sum(xp * xp, axis=1, keepdims=True))
    xn = xp / jnp.clip(norm, 1e-12, None)
    cbt = cbt_ref[...]
    csq = jnp.sum(cbt * cbt, axis=0, keepdims=True)
    dots2 = lax.dot_general(
        xn * -2.0, cbt, (((1,), (0,)), ((), ())),
        preferred_element_type=jnp.float32,
    )
    s = dots2 + csq
    out_ref[...] = jnp.argmin(s, axis=1).astype(jnp.int32)[:, None]


def kernel(x, P, CB):
    x2 = x.reshape(_B * _T, _D)
    cbt = CB.T
    out = pl.pallas_call(
        _vq_body,
        grid=(_B * _T // _TB,),
        in_specs=[
            pl.BlockSpec((_TB, _D), lambda i: (i, 0)),
            pl.BlockSpec((_D, _CD), lambda i: (0, 0)),
            pl.BlockSpec((_CD, _V), lambda i: (0, 0)),
        ],
        out_specs=pl.BlockSpec((_TB, 1), lambda i: (i, 0)),
        out_shape=jax.ShapeDtypeStruct((_B * _T, 1), jnp.int32),
    )(x2, P, cbt)
    return out.reshape(_B, _T)
